# baseline (speedup 1.0000x reference)
import jax
import jax.numpy as jnp
from jax import lax
from jax.experimental import pallas as pl
from jax.experimental.pallas import tpu as pltpu

_B, _L, _D = 64, 577, 768
_N = _L - 1
_K = _L // 2


def _rank_body(s_ref, keep_ref, vals_ref):
    st = s_ref[0]
    sc = st.T
    col = lax.broadcasted_iota(jnp.int32, (_N, _N), 1)
    row = lax.broadcasted_iota(jnp.int32, (_N, _N), 0)
    m = (st > sc) | ((st == sc) & (col < row))
    mb = m.astype(jnp.bfloat16)
    dn = (((1,), (0,)), ((), ()))
    ones = jnp.ones((_N, 1), jnp.bfloat16)
    rank = lax.dot_general(mb, ones, dn,
                           preferred_element_type=jnp.float32).astype(jnp.int32)

    colk = lax.broadcasted_iota(jnp.int32, (_N, _K), 1)
    eqb = (rank == colk).astype(jnp.bfloat16)
    ii = lax.broadcasted_iota(jnp.int32, (1, _N), 1)
    ihi = (ii // 256).astype(jnp.bfloat16)
    ilo = (ii % 256).astype(jnp.bfloat16)
    keep = (256.0 * lax.dot_general(ihi, eqb, dn, preferred_element_type=jnp.float32)
            + lax.dot_general(ilo, eqb, dn, preferred_element_type=jnp.float32))
    keep_ref[...] = keep.astype(jnp.int32)[None]

    hi = st.astype(jnp.bfloat16)
    r1 = st - hi.astype(jnp.float32)
    mid = r1.astype(jnp.bfloat16)
    lo = (r1 - mid.astype(jnp.float32)).astype(jnp.bfloat16)
    vals = (lax.dot_general(hi, eqb, dn, preferred_element_type=jnp.float32)
            + lax.dot_general(mid, eqb, dn, preferred_element_type=jnp.float32)
            + lax.dot_general(lo, eqb, dn, preferred_element_type=jnp.float32))
    vals_ref[...] = vals[None]


def _rank(scores):
    return pl.pallas_call(
        _rank_body,
        grid=(_B,),
        in_specs=[pl.BlockSpec((1, 1, _N), lambda b: (b, 0, 0))],
        out_specs=[
            pl.BlockSpec((1, 1, _K), lambda b: (b, 0, 0)),
            pl.BlockSpec((1, 1, _K), lambda b: (b, 0, 0)),
        ],
        out_shape=[
            jax.ShapeDtypeStruct((_B, 1, _K), jnp.int32),
            jax.ShapeDtypeStruct((_B, 1, _K), jnp.float32),
        ],
        compiler_params=pltpu.CompilerParams(dimension_semantics=("parallel",)),
    )(scores.reshape(_B, 1, _N))


def _cls_body(cls_ref, g_ref, out_ref):
    out_ref[...] = g_ref[...]
    out_ref[0, 0:1, :] = cls_ref[0]


def _fix_cls(cls_states, gathered):
    return pl.pallas_call(
        _cls_body,
        grid=(_B,),
        in_specs=[
            pl.BlockSpec((1, 1, _D), lambda b: (b, 0, 0)),
            pl.BlockSpec((1, 8, _D), lambda b: (b, 0, 0)),
        ],
        out_specs=[pl.BlockSpec((1, 8, _D), lambda b: (b, 0, 0))],
        out_shape=[jax.ShapeDtypeStruct((_B, _K + 1, _D), jnp.float32)],
        input_output_aliases={1: 0},
        compiler_params=pltpu.CompilerParams(dimension_semantics=("parallel",)),
    )(cls_states, gathered)[0]


def kernel(layer_idx, text_states, text_mask, image_states, image_mask,
           cross_attn, previous_keep_mask, ln_gamma, ln_beta, W, b):
    image_states_no_cls = image_states[:, 1:]
    x = image_states_no_cls
    mu = jnp.mean(x, axis=-1, keepdims=True)
    var = jnp.var(x, axis=-1, keepdims=True)
    normed = (x - mu) / jnp.sqrt(var + 1e-5) * ln_gamma + ln_beta
    logits = normed @ W + b
    token_scores = jax.nn.log_softmax(logits, axis=-1)
    scores = token_scores[:, :, 0]

    keep_idx, topk_vals = _rank(scores)
    keep_idx = keep_idx.reshape(_B, _K)
    topk_vals = topk_vals.reshape(_B, _K)

    idx289 = jnp.concatenate([jnp.zeros((_B, 1), jnp.int32), keep_idx], axis=1)
    gathered = jnp.take_along_axis(image_states_no_cls, idx289[:, :, None],
                                   axis=1)
    new_img_states = _fix_cls(image_states[:, :1], gathered)
    new_img_mask = jnp.ones((_B, _K + 1), jnp.int32)
    return (new_img_states, new_img_mask, keep_idx, scores, topk_vals)


# speedup vs baseline: 1.1008x; 1.1008x over previous
import jax
import jax.numpy as jnp
from jax import lax
from jax.experimental import pallas as pl
from jax.experimental.pallas import tpu as pltpu

_B, _L, _D = 64, 577, 768
_N = _L - 1
_K = _L // 2


def _rank_body(s_ref, keep_ref, vals_ref):
    st = s_ref[0]
    sc = st.T
    col = lax.broadcasted_iota(jnp.int32, (_N, _N), 1)
    row = lax.broadcasted_iota(jnp.int32, (_N, _N), 0)
    m = (st > sc) | ((st == sc) & (col < row))
    rank = jnp.sum(m.astype(jnp.int32), axis=1, keepdims=True)

    colk = lax.broadcasted_iota(jnp.int32, (_N, _K), 1)
    rowk = lax.broadcasted_iota(jnp.int32, (_N, _K), 0)
    eq = rank == colk
    keep_ref[...] = jnp.sum(jnp.where(eq, rowk, 0), axis=0, keepdims=True)[None]
    vals_ref[...] = jnp.sum(jnp.where(eq, sc, 0.0), axis=0, keepdims=True)[None]


def _rank(scores):
    return pl.pallas_call(
        _rank_body,
        grid=(_B,),
        in_specs=[pl.BlockSpec((1, 1, _N), lambda b: (b, 0, 0))],
        out_specs=[
            pl.BlockSpec((1, 1, _K), lambda b: (b, 0, 0)),
            pl.BlockSpec((1, 1, _K), lambda b: (b, 0, 0)),
        ],
        out_shape=[
            jax.ShapeDtypeStruct((_B, 1, _K), jnp.int32),
            jax.ShapeDtypeStruct((_B, 1, _K), jnp.float32),
        ],
        compiler_params=pltpu.CompilerParams(dimension_semantics=("parallel",)),
    )(scores.reshape(_B, 1, _N))


def _cls_body(cls_ref, g_ref, out_ref):
    out_ref[...] = g_ref[...]
    out_ref[0, 0:1, :] = cls_ref[0]


def _fix_cls(cls_states, gathered):
    return pl.pallas_call(
        _cls_body,
        grid=(_B,),
        in_specs=[
            pl.BlockSpec((1, 1, _D), lambda b: (b, 0, 0)),
            pl.BlockSpec((1, 8, _D), lambda b: (b, 0, 0)),
        ],
        out_specs=[pl.BlockSpec((1, 8, _D), lambda b: (b, 0, 0))],
        out_shape=[jax.ShapeDtypeStruct((_B, _K + 1, _D), jnp.float32)],
        input_output_aliases={1: 0},
        compiler_params=pltpu.CompilerParams(dimension_semantics=("parallel",)),
    )(cls_states, gathered)[0]


def kernel(layer_idx, text_states, text_mask, image_states, image_mask,
           cross_attn, previous_keep_mask, ln_gamma, ln_beta, W, b):
    image_states_no_cls = image_states[:, 1:]
    x = image_states_no_cls
    mu = jnp.mean(x, axis=-1, keepdims=True)
    var = jnp.var(x, axis=-1, keepdims=True)
    normed = (x - mu) / jnp.sqrt(var + 1e-5) * ln_gamma + ln_beta
    logits = normed @ W + b
    token_scores = jax.nn.log_softmax(logits, axis=-1)
    scores = token_scores[:, :, 0]

    keep_idx, topk_vals = _rank(scores)
    keep_idx = keep_idx.reshape(_B, _K)
    topk_vals = topk_vals.reshape(_B, _K)

    idx289 = jnp.concatenate([jnp.zeros((_B, 1), jnp.int32), keep_idx], axis=1)
    gathered = jnp.take_along_axis(image_states_no_cls, idx289[:, :, None],
                                   axis=1)
    new_img_states = _fix_cls(image_states[:, :1], gathered)
    new_img_mask = jnp.ones((_B, _K + 1), jnp.int32)
    return (new_img_states, new_img_mask, keep_idx, scores, topk_vals)


# speedup vs baseline: 1.3509x; 1.2272x over previous
import jax
import jax.numpy as jnp
from jax import lax
from jax.experimental import pallas as pl
from jax.experimental.pallas import tpu as pltpu

_B, _L, _D = 64, 577, 768
_N = _L - 1
_K = _L // 2


def _rank_body(s_ref, keep_ref, vals_ref):
    st = s_ref[0]
    sc = st.T
    col = lax.broadcasted_iota(jnp.int32, (_N, _N), 1)
    row = lax.broadcasted_iota(jnp.int32, (_N, _N), 0)
    m = (st > sc) | ((st == sc) & (col < row))
    rank = jnp.sum(m.astype(jnp.int32), axis=1, keepdims=True)

    colk = lax.broadcasted_iota(jnp.int32, (_N, _K), 1)
    rowk = lax.broadcasted_iota(jnp.int32, (_N, _K), 0)
    eq = rank == colk
    keep_ref[...] = jnp.sum(jnp.where(eq, rowk, 0), axis=0, keepdims=True)[None]
    vals_ref[...] = jnp.sum(jnp.where(eq, sc, 0.0), axis=0, keepdims=True)[None]


def _rank(scores):
    return pl.pallas_call(
        _rank_body,
        grid=(_B,),
        in_specs=[pl.BlockSpec((1, 1, _N), lambda b: (b, 0, 0))],
        out_specs=[
            pl.BlockSpec((1, 1, _K), lambda b: (b, 0, 0)),
            pl.BlockSpec((1, 1, _K), lambda b: (b, 0, 0)),
        ],
        out_shape=[
            jax.ShapeDtypeStruct((_B, 1, _K), jnp.int32),
            jax.ShapeDtypeStruct((_B, 1, _K), jnp.float32),
        ],
        compiler_params=pltpu.CompilerParams(dimension_semantics=("parallel",)),
    )(scores.reshape(_B, 1, _N))


def _cls_body(cls_ref, g_ref, out_ref):
    out_ref[...] = g_ref[...]
    out_ref[0, 0:1, :] = cls_ref[0]


def _fix_cls(cls_states, gathered):
    return pl.pallas_call(
        _cls_body,
        grid=(_B,),
        in_specs=[
            pl.BlockSpec((1, 1, _D), lambda b: (b, 0, 0)),
            pl.BlockSpec((1, 8, _D), lambda b: (b, 0, 0)),
        ],
        out_specs=[pl.BlockSpec((1, 8, _D), lambda b: (b, 0, 0))],
        out_shape=[jax.ShapeDtypeStruct((_B, _K + 1, _D), jnp.float32)],
        input_output_aliases={1: 0},
        compiler_params=pltpu.CompilerParams(dimension_semantics=("parallel",)),
    )(cls_states, gathered)[0]


def kernel(layer_idx, text_states, text_mask, image_states, image_mask,
           cross_attn, previous_keep_mask, ln_gamma, ln_beta, W, b):
    image_states_no_cls = image_states[:, 1:]
    x = image_states_no_cls
    mu = jnp.mean(x, axis=-1, keepdims=True)
    var = jnp.var(x, axis=-1, keepdims=True)
    normed = (x - mu) / jnp.sqrt(var + 1e-5) * ln_gamma + ln_beta
    logits = normed @ W + b
    token_scores = jax.nn.log_softmax(logits, axis=-1)
    scores = token_scores[:, :, 0]

    keep_idx, topk_vals = _rank(scores)
    keep_idx = keep_idx.reshape(_B, _K)
    topk_vals = topk_vals.reshape(_B, _K)

    gathered = jnp.take_along_axis(image_states_no_cls, keep_idx[:, :, None],
                                   axis=1)
    new_img_states = jnp.concatenate([image_states[:, :1], gathered], axis=1)
    new_img_mask = jnp.ones((_B, _K + 1), jnp.int32)
    return (new_img_states, new_img_mask, keep_idx, scores, topk_vals)


# speedup vs baseline: 1.3845x; 1.0249x over previous
import jax
import jax.numpy as jnp
from jax import lax
from jax.experimental import pallas as pl
from jax.experimental.pallas import tpu as pltpu

_B, _L, _D = 64, 577, 768
_N = _L - 1
_K = _L // 2


_RB = 4


def _rank_body(s_ref, keep_ref, vals_ref):
    col = lax.broadcasted_iota(jnp.int32, (_N, _N), 1)
    row = lax.broadcasted_iota(jnp.int32, (_N, _N), 0)
    tri = col < row
    colk = lax.broadcasted_iota(jnp.int32, (_N, _K), 1)
    rowk = lax.broadcasted_iota(jnp.int32, (_N, _K), 0)
    for k in range(_RB):
        st = s_ref[k]
        sc = st.T
        m = (st > sc) | ((st == sc) & tri)
        rank = jnp.sum(m.astype(jnp.int32), axis=1, keepdims=True)
        eq = rank == colk
        keep_ref[k] = jnp.sum(jnp.where(eq, rowk, 0), axis=0, keepdims=True)
        vals_ref[k] = jnp.sum(jnp.where(eq, sc, 0.0), axis=0, keepdims=True)


def _rank(scores):
    return pl.pallas_call(
        _rank_body,
        grid=(_B // _RB,),
        in_specs=[pl.BlockSpec((_RB, 1, _N), lambda b: (b, 0, 0))],
        out_specs=[
            pl.BlockSpec((_RB, 1, _K), lambda b: (b, 0, 0)),
            pl.BlockSpec((_RB, 1, _K), lambda b: (b, 0, 0)),
        ],
        out_shape=[
            jax.ShapeDtypeStruct((_B, 1, _K), jnp.int32),
            jax.ShapeDtypeStruct((_B, 1, _K), jnp.float32),
        ],
        compiler_params=pltpu.CompilerParams(dimension_semantics=("parallel",)),
    )(scores.reshape(_B, 1, _N))


def _cls_body(cls_ref, g_ref, out_ref):
    out_ref[...] = g_ref[...]
    out_ref[0, 0:1, :] = cls_ref[0]


def _fix_cls(cls_states, gathered):
    return pl.pallas_call(
        _cls_body,
        grid=(_B,),
        in_specs=[
            pl.BlockSpec((1, 1, _D), lambda b: (b, 0, 0)),
            pl.BlockSpec((1, 8, _D), lambda b: (b, 0, 0)),
        ],
        out_specs=[pl.BlockSpec((1, 8, _D), lambda b: (b, 0, 0))],
        out_shape=[jax.ShapeDtypeStruct((_B, _K + 1, _D), jnp.float32)],
        input_output_aliases={1: 0},
        compiler_params=pltpu.CompilerParams(dimension_semantics=("parallel",)),
    )(cls_states, gathered)[0]


def kernel(layer_idx, text_states, text_mask, image_states, image_mask,
           cross_attn, previous_keep_mask, ln_gamma, ln_beta, W, b):
    image_states_no_cls = image_states[:, 1:]
    x = image_states_no_cls
    mu = jnp.mean(x, axis=-1, keepdims=True)
    var = jnp.var(x, axis=-1, keepdims=True)
    normed = (x - mu) / jnp.sqrt(var + 1e-5) * ln_gamma + ln_beta
    logits = normed @ W + b
    token_scores = jax.nn.log_softmax(logits, axis=-1)
    scores = token_scores[:, :, 0]

    keep_idx, topk_vals = _rank(scores)
    keep_idx = keep_idx.reshape(_B, _K)
    topk_vals = topk_vals.reshape(_B, _K)

    gathered = jnp.take_along_axis(image_states_no_cls, keep_idx[:, :, None],
                                   axis=1)
    new_img_states = jnp.concatenate([image_states[:, :1], gathered], axis=1)
    new_img_mask = jnp.ones((_B, _K + 1), jnp.int32)
    return (new_img_states, new_img_mask, keep_idx, scores, topk_vals)
